# baseline (speedup 1.0000x reference)
import functools
import math

import jax
import jax.numpy as jnp
from jax.experimental import pallas as pl
from jax.experimental.pallas import tpu as pltpu

B = 16
D = 512
H = 256
NLAGS = 3
K = 16
TEMP = 5.0
SCALE = 1.0 / (math.sqrt(H) * TEMP)


def _oem_sort_pairs(n):
    pairs = []

    def merge(lo, m, r):
        step = r * 2
        if step < m:
            merge(lo, m, step)
            merge(lo + r, m, step)
            for i in range(lo + r, lo + m - r, step):
                pairs.append((i, i + r))
        else:
            pairs.append((lo, lo + r))

    def sort(lo, m):
        if m > 1:
            h = m // 2
            sort(lo, h)
            sort(lo + h, h)
            merge(lo, m, 1)

    sort(0, n)
    return pairs


def _bitonic_clean_pairs(n):
    pairs = []
    r = n // 2
    while r >= 1:
        for i in range(n):
            if i % (2 * r) < r:
                pairs.append((i, i + r))
        r //= 2
    return pairs


_OEM16 = _oem_sort_pairs(16)
_CLEAN16 = _bitonic_clean_pairs(16)


def _kth_largest_per_column(w):
    nb = D // K
    S = [w[s * nb:(s + 1) * nb, :] for s in range(K)]
    for (i, j) in _OEM16:
        hi = jnp.maximum(S[i], S[j])
        lo = jnp.minimum(S[i], S[j])
        S[i], S[j] = hi, lo
    rows = nb
    while rows > 1:
        h = rows // 2
        A = [x[:h] for x in S]
        Bv = [x[h:] for x in S]
        S = [jnp.maximum(A[s], Bv[K - 1 - s]) for s in range(K)]
        if h > 1:
            for (i, j) in _CLEAN16:
                hi = jnp.maximum(S[i], S[j])
                lo = jnp.minimum(S[i], S[j])
                S[i], S[j] = hi, lo
        rows = h
    t = S[0]
    for s in range(1, K):
        t = jnp.minimum(t, S[s])
    return t


BB = 4


def _body(zs_ref, w_ref, b_ref, bil_ref, adj_ref, logits_ref,
          proj_scr, prev_scr):
    lag = pl.program_id(1)

    @pl.when(lag == 0)
    def _compute_proj():
        for bb in range(BB):
            acc = jax.lax.dot_general(
                zs_ref[bb], w_ref[...],
                dimension_numbers=(((1,), (1,)), ((), ())),
                preferred_element_type=jnp.float32)
            proj_scr[bb] = jnp.tanh(acc + b_ref[...])
        prev_scr[...] = jnp.full((BB, D, D), jnp.inf, jnp.float32)

    rows = jax.lax.broadcasted_iota(jnp.int32, (D, D), 0)
    cols = jax.lax.broadcasted_iota(jnp.int32, (D, D), 1)
    for bb in range(BB):
        proj = proj_scr[bb]
        projl = jax.lax.dot_general(
            proj, bil_ref[0],
            dimension_numbers=(((1,), (0,)), ((), ())),
            preferred_element_type=jnp.float32)
        logits = jax.lax.dot_general(
            projl, proj,
            dimension_numbers=(((1,), (1,)), ((), ())),
            preferred_element_type=jnp.float32) * SCALE
        logits = jnp.where(rows == cols, 0.0, logits)
        logits_ref[0, bb] = logits

        t = _kth_largest_per_column(logits)
        adj = jnp.where(logits >= t, jnp.maximum(logits, 0.0), 0.0)

        chained = jnp.minimum(adj, prev_scr[bb])
        prev_scr[bb] = chained
        adj_ref[0, bb] = chained


@jax.jit
def kernel(ZS, W, b, bilinear):
    b2d = b.reshape(1, H)
    grid = (B // BB, NLAGS)
    out_shape = (
        jax.ShapeDtypeStruct((NLAGS, B, D, D), jnp.float32),
        jax.ShapeDtypeStruct((NLAGS, B, D, D), jnp.float32),
    )
    adj, logits = pl.pallas_call(
        _body,
        grid=grid,
        in_specs=[
            pl.BlockSpec((BB, D, D), lambda bi, l: (bi, 0, 0)),
            pl.BlockSpec((H, D), lambda bi, l: (0, 0)),
            pl.BlockSpec((1, H), lambda bi, l: (0, 0)),
            pl.BlockSpec((1, H, H), lambda bi, l: (l, 0, 0)),
        ],
        out_specs=(
            pl.BlockSpec((1, BB, D, D), lambda bi, l: (l, bi, 0, 0)),
            pl.BlockSpec((1, BB, D, D), lambda bi, l: (l, bi, 0, 0)),
        ),
        scratch_shapes=[
            pltpu.VMEM((BB, D, H), jnp.float32),
            pltpu.VMEM((BB, D, D), jnp.float32),
        ],
        out_shape=out_shape,
    )(ZS, W, b2d, bilinear)
    return adj, logits


# speedup vs baseline: 1.0457x; 1.0457x over previous
import functools
import math

import jax
import jax.numpy as jnp
from jax.experimental import pallas as pl
from jax.experimental.pallas import tpu as pltpu

B = 16
D = 512
H = 256
NLAGS = 3
K = 16
TEMP = 5.0
SCALE = 1.0 / (math.sqrt(H) * TEMP)


def _oem_sort_pairs(n):
    pairs = []

    def merge(lo, m, r):
        step = r * 2
        if step < m:
            merge(lo, m, step)
            merge(lo + r, m, step)
            for i in range(lo + r, lo + m - r, step):
                pairs.append((i, i + r))
        else:
            pairs.append((lo, lo + r))

    def sort(lo, m):
        if m > 1:
            h = m // 2
            sort(lo, h)
            sort(lo + h, h)
            merge(lo, m, 1)

    sort(0, n)
    return pairs


def _bitonic_clean_pairs(n):
    pairs = []
    r = n // 2
    while r >= 1:
        for i in range(n):
            if i % (2 * r) < r:
                pairs.append((i, i + r))
        r //= 2
    return pairs


_OEM16 = _oem_sort_pairs(16)
_CLEAN16 = _bitonic_clean_pairs(16)


def _kth_largest_per_column(w):
    nb = D // K
    S = [w[s * nb:(s + 1) * nb, :] for s in range(K)]
    for (i, j) in _OEM16:
        hi = jnp.maximum(S[i], S[j])
        lo = jnp.minimum(S[i], S[j])
        S[i], S[j] = hi, lo
    rows = nb
    while rows > 1:
        h = rows // 2
        A = [x[:h] for x in S]
        Bv = [x[h:] for x in S]
        S = [jnp.maximum(A[s], Bv[K - 1 - s]) for s in range(K)]
        if h > 1:
            for (i, j) in _CLEAN16:
                hi = jnp.maximum(S[i], S[j])
                lo = jnp.minimum(S[i], S[j])
                S[i], S[j] = hi, lo
        rows = h
    t = S[0]
    for s in range(1, K):
        t = jnp.minimum(t, S[s])
    return t


BB = 4


def _body(zs_ref, w_ref, b_ref, bil_ref, adj_ref, logits_ref,
          proj_scr, prev_scr):
    lag = pl.program_id(1)

    @pl.when(lag == 0)
    def _compute_proj():
        for bb in range(BB):
            acc = jax.lax.dot_general(
                zs_ref[bb], w_ref[...],
                dimension_numbers=(((1,), (1,)), ((), ())),
                preferred_element_type=jnp.float32)
            proj_scr[bb] = jnp.tanh(acc + b_ref[...])
        prev_scr[...] = jnp.full((BB, D, D), jnp.inf, jnp.float32)

    rows = jax.lax.broadcasted_iota(jnp.int32, (D, D), 0)
    cols = jax.lax.broadcasted_iota(jnp.int32, (D, D), 1)
    for bb in range(BB):
        proj = proj_scr[bb]
        projl = jax.lax.dot_general(
            proj, bil_ref[0],
            dimension_numbers=(((1,), (0,)), ((), ())),
            preferred_element_type=jnp.float32)
        logits = jax.lax.dot_general(
            projl, proj,
            dimension_numbers=(((1,), (1,)), ((), ())),
            preferred_element_type=jnp.float32) * SCALE
        logits = jnp.where(rows == cols, 0.0, logits)
        logits_ref[0, bb] = logits

        w = jnp.maximum(logits, 0.0)
        t = _kth_largest_per_column(w)
        adj = jnp.where(w >= t, w, 0.0)

        chained = jnp.minimum(adj, prev_scr[bb])
        prev_scr[bb] = chained
        adj_ref[0, bb] = chained


@jax.jit
def kernel(ZS, W, b, bilinear):
    b2d = b.reshape(1, H)
    grid = (B // BB, NLAGS)
    out_shape = (
        jax.ShapeDtypeStruct((NLAGS, B, D, D), jnp.float32),
        jax.ShapeDtypeStruct((NLAGS, B, D, D), jnp.float32),
    )
    adj, logits = pl.pallas_call(
        _body,
        grid=grid,
        in_specs=[
            pl.BlockSpec((BB, D, D), lambda bi, l: (bi, 0, 0)),
            pl.BlockSpec((H, D), lambda bi, l: (0, 0)),
            pl.BlockSpec((1, H), lambda bi, l: (0, 0)),
            pl.BlockSpec((1, H, H), lambda bi, l: (l, 0, 0)),
        ],
        out_specs=(
            pl.BlockSpec((1, BB, D, D), lambda bi, l: (l, bi, 0, 0)),
            pl.BlockSpec((1, BB, D, D), lambda bi, l: (l, bi, 0, 0)),
        ),
        scratch_shapes=[
            pltpu.VMEM((BB, D, H), jnp.float32),
            pltpu.VMEM((BB, D, D), jnp.float32),
        ],
        out_shape=out_shape,
    )(ZS, W, b2d, bilinear)
    return adj, logits
